# Initial kernel scaffold; baseline (speedup 1.0000x reference)
import jax
import jax.numpy as jnp
from jax.experimental import pallas as pl


def kernel(x, seeds):
    raise NotImplementedError("write your pallas kernel here")


# speedup vs baseline: 1.5668x; 1.5668x over previous
import jax
import jax.numpy as jnp
from jax.experimental import pallas as pl
from jax.experimental.pallas import tpu as pltpu

_B = 32
_S = 8192
_D = 512
_M = 16
_CHUNK = 2048
_NC = _S // _CHUNK
_NEG_BIG = -1e30


def _ap_kernel(x_ref, seeds_ref, o_ref, acc_ref, m_ref, l_ref):
    j = pl.program_id(1)

    @pl.when(j == 0)
    def _init():
        m_ref[...] = jnp.full_like(m_ref, _NEG_BIG)
        l_ref[...] = jnp.zeros_like(l_ref)
        acc_ref[...] = jnp.zeros_like(acc_ref)

    xb = x_ref[0]
    s = jax.lax.dot_general(
        seeds_ref[...], xb,
        dimension_numbers=(((1,), (1,)), ((), ())),
        preferred_element_type=jnp.float32,
    )
    cmax = jnp.max(s, axis=1, keepdims=True)
    m_old = m_ref[:, 0:1]
    m_new = jnp.maximum(m_old, cmax)
    alpha = jnp.exp(m_old - m_new)
    p = jnp.exp(s - m_new)
    l_ref[...] = l_ref[...] * alpha + jnp.broadcast_to(
        jnp.sum(p, axis=1, keepdims=True), l_ref.shape)
    pv = jax.lax.dot_general(
        p, xb,
        dimension_numbers=(((1,), (0,)), ((), ())),
        preferred_element_type=jnp.float32,
    )
    acc_ref[...] = acc_ref[...] * alpha + pv
    m_ref[...] = jnp.broadcast_to(m_new, m_ref.shape)

    @pl.when(j == _NC - 1)
    def _finish():
        o_ref[0] = acc_ref[...] / l_ref[:, 0:1]


@jax.jit
def kernel(x, seeds):
    return pl.pallas_call(
        _ap_kernel,
        grid=(_B, _NC),
        in_specs=[
            pl.BlockSpec((1, _CHUNK, _D), lambda b, j: (b, j, 0)),
            pl.BlockSpec((_M, _D), lambda b, j: (0, 0)),
        ],
        out_specs=pl.BlockSpec((1, _M, _D), lambda b, j: (b, 0, 0)),
        out_shape=jax.ShapeDtypeStruct((_B, _M, _D), jnp.float32),
        scratch_shapes=[
            pltpu.VMEM((_M, _D), jnp.float32),
            pltpu.VMEM((_M, 128), jnp.float32),
            pltpu.VMEM((_M, 128), jnp.float32),
        ],
        compiler_params=pltpu.CompilerParams(
            dimension_semantics=("parallel", "arbitrary"),
        ),
        name="ap_pool",
    )(x, seeds)


# speedup vs baseline: 1.8601x; 1.1872x over previous
import jax
import jax.numpy as jnp
from jax.experimental import pallas as pl
from jax.experimental.pallas import tpu as pltpu

_B = 32
_S = 8192
_D = 512
_M = 16
_CHUNK = 4096
_NC = _S // _CHUNK
_NEG_BIG = -1e30


def _ap_kernel(x_ref, seeds_ref, o_ref, acc_ref, m_ref, l_ref):
    j = pl.program_id(1)

    @pl.when(j == 0)
    def _init():
        m_ref[...] = jnp.full_like(m_ref, _NEG_BIG)
        l_ref[...] = jnp.zeros_like(l_ref)
        acc_ref[...] = jnp.zeros_like(acc_ref)

    xb = x_ref[0]
    s = jax.lax.dot_general(
        seeds_ref[...], xb,
        dimension_numbers=(((1,), (1,)), ((), ())),
        preferred_element_type=jnp.float32,
    )
    cmax = jnp.max(s, axis=1, keepdims=True)
    m_old = m_ref[:, 0:1]
    m_new = jnp.maximum(m_old, cmax)
    alpha = jnp.exp(m_old - m_new)
    p = jnp.exp(s - m_new)
    l_ref[...] = l_ref[...] * alpha + jnp.broadcast_to(
        jnp.sum(p, axis=1, keepdims=True), l_ref.shape)
    pv = jax.lax.dot_general(
        p, xb,
        dimension_numbers=(((1,), (0,)), ((), ())),
        preferred_element_type=jnp.float32,
    )
    acc_ref[...] = acc_ref[...] * alpha + pv
    m_ref[...] = jnp.broadcast_to(m_new, m_ref.shape)

    @pl.when(j == _NC - 1)
    def _finish():
        o_ref[0] = acc_ref[...] / l_ref[:, 0:1]


@jax.jit
def kernel(x, seeds):
    return pl.pallas_call(
        _ap_kernel,
        grid=(_B, _NC),
        in_specs=[
            pl.BlockSpec((1, _CHUNK, _D), lambda b, j: (b, j, 0)),
            pl.BlockSpec((_M, _D), lambda b, j: (0, 0)),
        ],
        out_specs=pl.BlockSpec((1, _M, _D), lambda b, j: (b, 0, 0)),
        out_shape=jax.ShapeDtypeStruct((_B, _M, _D), jnp.float32),
        scratch_shapes=[
            pltpu.VMEM((_M, _D), jnp.float32),
            pltpu.VMEM((_M, 128), jnp.float32),
            pltpu.VMEM((_M, 128), jnp.float32),
        ],
        compiler_params=pltpu.CompilerParams(
            dimension_semantics=("parallel", "arbitrary"),
        ),
        name="ap_pool",
    )(x, seeds)
